# Initial kernel scaffold; baseline (speedup 1.0000x reference)
import jax
import jax.numpy as jnp
from jax.experimental import pallas as pl


def kernel(seq_len, relative_attention_bias):
    raise NotImplementedError("write your pallas kernel here")


# speedup vs baseline: 36.3658x; 36.3658x over previous
import functools

import jax
import jax.numpy as jnp
from jax import lax
from jax.experimental import pallas as pl
from jax.experimental.pallas import tpu as pltpu
from jax.experimental.pallas import tpu_sc as plsc

N_HEADS = 16
NUM_BUCKETS = 32
SEQ_LEN = 2048
_D_MAX = SEQ_LEN - 1
_NVALS = 2 * SEQ_LEN - 1
_W = 4096
_GH = 16
_THRESH = (12, 16, 23, 32, 46, 64, 91)


def _body(table_hbm, out_hbm, table_v, vals_v, b_v):
    head = lax.axis_index("s")
    half = lax.axis_index("c")
    lane = lax.iota(jnp.int32, 16)

    pltpu.sync_copy(table_hbm, table_v)

    def vals_chunk(c, _):
        m = lane + c * 16
        d = jnp.minimum(m, _NVALS - 1) - _D_MAX
        a = jnp.abs(d)
        big = jnp.full((16,), 8, jnp.int32)
        for t in _THRESH:
            big = big + (a >= t).astype(jnp.int32)
        bkt = jnp.where(a < 8, a, big) + jnp.where(d > 0, 16, 0)
        vals_v[pl.ds(c * 16, 16)] = plsc.load_gather(
            table_v, [bkt * N_HEADS + head])
        return 0

    lax.fori_loop(0, _W // 16, vals_chunk, 0, unroll=False)

    for r in range(_GH):
        def b_chunk(c, _, r=r):
            idx = jnp.minimum(lane + c * 16 + (_GH - 1 - r), _NVALS - 1)
            b_v[pl.ds(r * _W + c * 16, 16)] = plsc.load_gather(vals_v, [idx])
            return 0

        lax.fori_loop(0, _W // 16, b_chunk, 0, unroll=False)

    def emit(g, _):
        i = half * (SEQ_LEN // 2) + g * _GH
        c0 = SEQ_LEN - _GH - i
        rowbase = head * SEQ_LEN + i
        for r in range(_GH):
            pltpu.sync_copy(
                b_v.at[pl.ds(r * _W + c0, SEQ_LEN)],
                out_hbm.at[pl.ds((rowbase + r) * SEQ_LEN, SEQ_LEN)])
        return 0

    lax.fori_loop(0, SEQ_LEN // 2 // _GH, emit, 0, unroll=False)


@jax.jit
def _bias_flat(table_flat):
    mesh = plsc.VectorSubcoreMesh(core_axis_name="c", subcore_axis_name="s")
    return pl.kernel(
        _body,
        mesh=mesh,
        compiler_params=pltpu.CompilerParams(needs_layout_passes=False),
        out_type=jax.ShapeDtypeStruct((N_HEADS * SEQ_LEN * SEQ_LEN,),
                                      jnp.float32),
        scratch_types=[
            pltpu.VMEM((NUM_BUCKETS * N_HEADS,), jnp.float32),
            pltpu.VMEM((_W,), jnp.float32),
            pltpu.VMEM((_GH * _W,), jnp.float32),
        ],
    )(table_flat)


def kernel(seq_len, relative_attention_bias):
    del seq_len
    table_flat = relative_attention_bias.reshape(-1)
    out = _bias_flat(table_flat)
    return out.reshape(N_HEADS, SEQ_LEN, SEQ_LEN)


# speedup vs baseline: 41.6440x; 1.1451x over previous
import functools

import jax
import jax.numpy as jnp
from jax import lax
from jax.experimental import pallas as pl
from jax.experimental.pallas import tpu as pltpu
from jax.experimental.pallas import tpu_sc as plsc

N_HEADS = 16
NUM_BUCKETS = 32
SEQ_LEN = 2048
_D_MAX = SEQ_LEN - 1
_NVALS = 2 * SEQ_LEN - 1
_W = 4096
_GH = 16
_THRESH = (12, 16, 23, 32, 46, 64, 91)


def _body(table_hbm, out_hbm, table_v, vals_v, b_v, dma_sem):
    head = lax.axis_index("s")
    half = lax.axis_index("c")
    lane = lax.iota(jnp.int32, 16)

    pltpu.sync_copy(table_hbm, table_v)

    def vals_chunk(c, _):
        m = lane + c * 16
        d = jnp.minimum(m, _NVALS - 1) - _D_MAX
        a = jnp.abs(d)
        big = jnp.full((16,), 8, jnp.int32)
        for t in _THRESH:
            big = big + (a >= t).astype(jnp.int32)
        bkt = jnp.where(a < 8, a, big) + jnp.where(d > 0, 16, 0)
        vals_v[pl.ds(c * 16, 16)] = plsc.load_gather(
            table_v, [bkt * N_HEADS + head])
        return 0

    lax.fori_loop(0, _W // 16, vals_chunk, 0, unroll=False)

    for r in range(_GH):
        def b_chunk(c, _, r=r):
            idx = jnp.minimum(lane + c * 16 + (_GH - 1 - r), _NVALS - 1)
            b_v[pl.ds(r * _W + c * 16, 16)] = plsc.load_gather(vals_v, [idx])
            return 0

        lax.fori_loop(0, _W // 16, b_chunk, 0, unroll=False)

    def issue(g):
        i = half * (SEQ_LEN // 2) + g * _GH
        c0 = SEQ_LEN - _GH - i
        rowbase = head * SEQ_LEN + i
        for r in range(_GH):
            pltpu.async_copy(
                b_v.at[pl.ds(r * _W + c0, SEQ_LEN)],
                out_hbm.at[pl.ds((rowbase + r) * SEQ_LEN, SEQ_LEN)],
                dma_sem)

    def drain_group():
        for _ in range(_GH):
            pltpu.make_async_copy(
                b_v.at[pl.ds(0, SEQ_LEN)],
                out_hbm.at[pl.ds(head * SEQ_LEN * SEQ_LEN, SEQ_LEN)],
                dma_sem).wait()

    n_groups = SEQ_LEN // 2 // _GH
    issue(0)

    def emit(g, _):
        issue(g)
        drain_group()
        return 0

    lax.fori_loop(1, n_groups, emit, 0, unroll=False)
    drain_group()


@jax.jit
def _bias_flat(table_flat):
    mesh = plsc.VectorSubcoreMesh(core_axis_name="c", subcore_axis_name="s")
    return pl.kernel(
        _body,
        mesh=mesh,
        compiler_params=pltpu.CompilerParams(needs_layout_passes=False),
        out_type=jax.ShapeDtypeStruct((N_HEADS * SEQ_LEN * SEQ_LEN,),
                                      jnp.float32),
        scratch_types=[
            pltpu.VMEM((NUM_BUCKETS * N_HEADS,), jnp.float32),
            pltpu.VMEM((_W,), jnp.float32),
            pltpu.VMEM((_GH * _W,), jnp.float32),
            pltpu.SemaphoreType.DMA,
        ],
    )(table_flat)


def kernel(seq_len, relative_attention_bias):
    del seq_len
    table_flat = relative_attention_bias.reshape(-1)
    out = _bias_flat(table_flat)
    return out.reshape(N_HEADS, SEQ_LEN, SEQ_LEN)


# speedup vs baseline: 136.1876x; 3.2703x over previous
import jax
import jax.numpy as jnp
from jax import lax
from jax.experimental import pallas as pl
from jax.experimental.pallas import tpu as pltpu
from jax.experimental.pallas import tpu_sc as plsc

N_HEADS = 16
NUM_BUCKETS = 32
SEQ_LEN = 2048
_D_MAX = SEQ_LEN - 1
_NVALS = 2 * SEQ_LEN - 1
_BW = 31 * 128
_THRESH = (12, 16, 23, 32, 46, 64, 91)
_N_CLASSES_PER_SUBCORE = 8


def _body(table_hbm, out_hbm, table_v, vals_v, b2a, b2b, dma_sem):
    core = lax.axis_index("c")
    sub = lax.axis_index("s")
    wid = sub * 2 + core
    lane = lax.iota(jnp.int32, 16)

    pltpu.sync_copy(table_hbm, table_v)
    bufs = (b2a, b2b)

    def drain_class(buf):
        for _ in range(16):
            pltpu.make_async_copy(
                buf.at[:, pl.ds(0, SEQ_LEN)],
                out_hbm.at[0, pl.ds(0, 8), :],
                dma_sem).wait()

    for k in range(_N_CLASSES_PER_SUBCORE):
        cls = wid + 32 * k
        h = lax.shift_right_logical(cls, 4)
        m = lax.bitwise_and(cls, 15)
        qmin = 127 - 8 * m
        b2 = bufs[k % 2]

        if k >= 2:
            drain_class(b2)

        def vals_chunk(cc, _, h=h):
            x = lane + cc * 16
            d = jnp.minimum(x, _NVALS - 1) - _D_MAX
            a = jnp.abs(d)
            big = jnp.full((16,), 8, jnp.int32)
            for t in _THRESH:
                big = big + (a >= t).astype(jnp.int32)
            bkt = jnp.where(a < 8, a, big) + jnp.where(d > 0, 16, 0)
            vals_v[pl.ds(cc * 16, 16)] = plsc.load_gather(
                table_v, [bkt * N_HEADS + h])
            return 0

        lax.fori_loop(0, (_NVALS + 1) // 16, vals_chunk, 0, unroll=False)

        for rr in range(8):
            def b_chunk(cc, _, rr=rr, qmin=qmin, b2=b2):
                idx = qmin + cc * 16 + lane - rr
                b2[rr, pl.ds(cc * 16, 16)] = plsc.load_gather(vals_v, [idx])
                return 0

            lax.fori_loop(0, _BW // 16, b_chunk, 0, unroll=False)

        for u in range(16):
            co = 1920 - 128 * u
            pltpu.async_copy(
                b2.at[:, pl.ds(co, SEQ_LEN)],
                out_hbm.at[h, pl.ds(8 * m + 128 * u, 8), :],
                dma_sem)

    drain_class(bufs[0])
    drain_class(bufs[1])


@jax.jit
def _bias_sc(table_flat):
    mesh = plsc.VectorSubcoreMesh(core_axis_name="c", subcore_axis_name="s")
    return pl.kernel(
        _body,
        mesh=mesh,
        compiler_params=pltpu.CompilerParams(needs_layout_passes=False),
        out_type=jax.ShapeDtypeStruct((N_HEADS, SEQ_LEN, SEQ_LEN),
                                      jnp.float32),
        scratch_types=[
            pltpu.VMEM((NUM_BUCKETS * N_HEADS,), jnp.float32),
            pltpu.VMEM((_NVALS + 1,), jnp.float32),
            pltpu.VMEM((8, _BW), jnp.float32),
            pltpu.VMEM((8, _BW), jnp.float32),
            pltpu.SemaphoreType.DMA,
        ],
    )(table_flat)


def kernel(seq_len, relative_attention_bias):
    del seq_len
    return _bias_sc(relative_attention_bias.reshape(-1))


# speedup vs baseline: 136.7400x; 1.0041x over previous
import jax
import jax.numpy as jnp
from jax import lax
from jax.experimental import pallas as pl
from jax.experimental.pallas import tpu as pltpu
from jax.experimental.pallas import tpu_sc as plsc

N_HEADS = 16
NUM_BUCKETS = 32
SEQ_LEN = 2048
_D_MAX = SEQ_LEN - 1
_NVALS = 2 * SEQ_LEN - 1
_BW = 31 * 128
_THRESH = (12, 16, 23, 32, 46, 64, 91)
_N_CLASSES_PER_SUBCORE = 8


def _body(table_hbm, out_hbm, table_v, vals_v, b2a, b2b, b2c, dma_sem):
    core = lax.axis_index("c")
    sub = lax.axis_index("s")
    h = sub
    lane = lax.iota(jnp.int32, 16)

    pltpu.sync_copy(table_hbm, table_v)
    bufs = (b2a, b2b, b2c)

    def drain_class(buf):
        for _ in range(16):
            pltpu.make_async_copy(
                buf.at[:, pl.ds(0, SEQ_LEN)],
                out_hbm.at[0, pl.ds(0, 8), :],
                dma_sem).wait()

    def vals_chunk(cc, _):
        x = lane + cc * 16
        d = jnp.minimum(x, _NVALS - 1) - _D_MAX
        a = jnp.abs(d)
        big = jnp.full((16,), 8, jnp.int32)
        for t in _THRESH:
            big = big + (a >= t).astype(jnp.int32)
        bkt = jnp.where(a < 8, a, big) + jnp.where(d > 0, 16, 0)
        vals_v[pl.ds(cc * 16, 16)] = plsc.load_gather(
            table_v, [bkt * N_HEADS + h])
        return 0

    lax.fori_loop(0, (_NVALS + 1) // 16, vals_chunk, 0, unroll=False)

    for k in range(_N_CLASSES_PER_SUBCORE):
        m = core * 8 + k
        qmin = 127 - 8 * m
        b2 = bufs[k % 3]

        if k >= 3:
            drain_class(b2)

        for rr in range(8):
            def b_chunk(cc, _, rr=rr, qmin=qmin, b2=b2):
                idx = qmin + cc * 16 + lane - rr
                b2[rr, pl.ds(cc * 16, 16)] = plsc.load_gather(vals_v, [idx])
                return 0

            lax.fori_loop(0, _BW // 16, b_chunk, 0, unroll=False)

        for u in range(16):
            co = 1920 - 128 * u
            pltpu.async_copy(
                b2.at[:, pl.ds(co, SEQ_LEN)],
                out_hbm.at[h, pl.ds(8 * m + 128 * u, 8), :],
                dma_sem)

    for buf in bufs[len(bufs) - 3:]:
        drain_class(buf)


@jax.jit
def _bias_sc(table_flat):
    mesh = plsc.VectorSubcoreMesh(core_axis_name="c", subcore_axis_name="s")
    return pl.kernel(
        _body,
        mesh=mesh,
        compiler_params=pltpu.CompilerParams(needs_layout_passes=False),
        out_type=jax.ShapeDtypeStruct((N_HEADS, SEQ_LEN, SEQ_LEN),
                                      jnp.float32),
        scratch_types=[
            pltpu.VMEM((NUM_BUCKETS * N_HEADS,), jnp.float32),
            pltpu.VMEM((_NVALS + 1,), jnp.float32),
            pltpu.VMEM((8, _BW), jnp.float32),
            pltpu.VMEM((8, _BW), jnp.float32),
            pltpu.VMEM((8, _BW), jnp.float32),
            pltpu.SemaphoreType.DMA,
        ],
    )(table_flat)


def kernel(seq_len, relative_attention_bias):
    del seq_len
    return _bias_sc(relative_attention_bias.reshape(-1))


# speedup vs baseline: 141.6195x; 1.0357x over previous
import jax
import jax.numpy as jnp
from jax import lax
from jax.experimental import pallas as pl
from jax.experimental.pallas import tpu as pltpu
from jax.experimental.pallas import tpu_sc as plsc

N_HEADS = 16
NUM_BUCKETS = 32
SEQ_LEN = 2048
_D_MAX = SEQ_LEN - 1
_NVALS = 2 * SEQ_LEN - 1
_BW = 31 * 128
_THRESH = (12, 16, 23, 32, 46, 64, 91)
_N_CLASSES_PER_SUBCORE = 8


def _body(table_hbm, out_hbm, table_v, vals_v, b2a, b2b, b2c, dma_sem):
    core = lax.axis_index("c")
    sub = lax.axis_index("s")
    h = sub
    lane = lax.iota(jnp.int32, 16)

    pltpu.sync_copy(table_hbm, table_v)
    bufs = (b2a, b2b, b2c)

    def drain_class(buf):
        for _ in range(16):
            pltpu.make_async_copy(
                buf.at[:, pl.ds(0, SEQ_LEN)],
                out_hbm.at[0, pl.ds(0, 8), :],
                dma_sem).wait()

    def vals_chunk(cc, _):
        x = lane + cc * 16
        d = jnp.minimum(x, _NVALS - 1) - _D_MAX
        a = jnp.abs(d)
        big = jnp.full((16,), 8, jnp.int32)
        for t in _THRESH:
            big = big + (a >= t).astype(jnp.int32)
        bkt = jnp.where(a < 8, a, big) + jnp.where(d > 0, 16, 0)
        vals_v[pl.ds(cc * 16, 16)] = plsc.load_gather(
            table_v, [bkt * N_HEADS + h])
        return 0

    lax.fori_loop(0, (_NVALS + 1) // 16, vals_chunk, 0, unroll=False)

    for k in range(_N_CLASSES_PER_SUBCORE):
        m = core * 8 + k
        qmin = 127 - 8 * m
        b2 = bufs[k % 3]

        if k >= 3:
            drain_class(b2)

        def b_chunk(cc, _, qmin=qmin, b2=b2):
            base = qmin + cc * 16
            for rr in range(8):
                b2[rr, pl.ds(cc * 16, 16)] = vals_v[pl.ds(base - rr, 16)]
            return 0

        lax.fori_loop(0, _BW // 16, b_chunk, 0, unroll=False)

        for u in range(16):
            co = 1920 - 128 * u
            pltpu.async_copy(
                b2.at[:, pl.ds(co, SEQ_LEN)],
                out_hbm.at[h, pl.ds(8 * m + 128 * u, 8), :],
                dma_sem)

    for buf in bufs[len(bufs) - 3:]:
        drain_class(buf)


@jax.jit
def _bias_sc(table_flat):
    mesh = plsc.VectorSubcoreMesh(core_axis_name="c", subcore_axis_name="s")
    return pl.kernel(
        _body,
        mesh=mesh,
        compiler_params=pltpu.CompilerParams(needs_layout_passes=False),
        out_type=jax.ShapeDtypeStruct((N_HEADS, SEQ_LEN, SEQ_LEN),
                                      jnp.float32),
        scratch_types=[
            pltpu.VMEM((NUM_BUCKETS * N_HEADS,), jnp.float32),
            pltpu.VMEM((_NVALS + 1,), jnp.float32),
            pltpu.VMEM((8, _BW), jnp.float32),
            pltpu.VMEM((8, _BW), jnp.float32),
            pltpu.VMEM((8, _BW), jnp.float32),
            pltpu.SemaphoreType.DMA,
        ],
    )(table_flat)


def kernel(seq_len, relative_attention_bias):
    del seq_len
    return _bias_sc(relative_attention_bias.reshape(-1))


# speedup vs baseline: 141.7181x; 1.0007x over previous
import jax
import jax.numpy as jnp
from jax import lax
from jax.experimental import pallas as pl
from jax.experimental.pallas import tpu as pltpu
from jax.experimental.pallas import tpu_sc as plsc

N_HEADS = 16
NUM_BUCKETS = 32
SEQ_LEN = 2048
_D_MAX = SEQ_LEN - 1
_NVALS = 2 * SEQ_LEN - 1
_BW = 31 * 128
_THRESH = (12, 16, 23, 32, 46, 64, 91)
_N_CLASSES_PER_SUBCORE = 8


def _body(table_hbm, out_hbm, table_v, vals_v, b2a, b2b, b2c, dma_sem):
    core = lax.axis_index("c")
    sub = lax.axis_index("s")
    h = sub
    lane = lax.iota(jnp.int32, 16)

    pltpu.sync_copy(table_hbm, table_v)
    bufs = (b2a, b2b, b2c)

    def drain_class(buf):
        for _ in range(16):
            pltpu.make_async_copy(
                buf.at[:, pl.ds(0, SEQ_LEN)],
                out_hbm.at[0, pl.ds(0, 8), :],
                dma_sem).wait()

    def vals_chunk(cc, _):
        x = lane + cc * 16
        d = jnp.minimum(x, _NVALS - 1) - _D_MAX
        a = jnp.abs(d)
        big = jnp.full((16,), 8, jnp.int32)
        for t in _THRESH:
            big = big + (a >= t).astype(jnp.int32)
        bkt = jnp.where(a < 8, a, big) + jnp.where(d > 0, 16, 0)
        col = jnp.full((16,), h, jnp.int32)
        vals_v[pl.ds(cc * 16, 16)] = plsc.load_gather(table_v, [bkt, col])
        return 0

    lax.fori_loop(0, (_NVALS + 1) // 16, vals_chunk, 0, unroll=False)

    for k in range(_N_CLASSES_PER_SUBCORE):
        m = core * 8 + k
        qmin = 127 - 8 * m
        b2 = bufs[k % 3]

        if k >= 3:
            drain_class(b2)

        def b_chunk(cc, _, qmin=qmin, b2=b2):
            base = qmin + cc * 16
            for rr in range(8):
                b2[rr, pl.ds(cc * 16, 16)] = vals_v[pl.ds(base - rr, 16)]
            return 0

        lax.fori_loop(0, _BW // 16, b_chunk, 0, unroll=False)

        for u in range(16):
            co = 1920 - 128 * u
            pltpu.async_copy(
                b2.at[:, pl.ds(co, SEQ_LEN)],
                out_hbm.at[h, pl.ds(8 * m + 128 * u, 8), :],
                dma_sem)

    for buf in bufs[len(bufs) - 3:]:
        drain_class(buf)


@jax.jit
def _bias_sc(table):
    mesh = plsc.VectorSubcoreMesh(core_axis_name="c", subcore_axis_name="s")
    return pl.kernel(
        _body,
        mesh=mesh,
        compiler_params=pltpu.CompilerParams(needs_layout_passes=False),
        out_type=jax.ShapeDtypeStruct((N_HEADS, SEQ_LEN, SEQ_LEN),
                                      jnp.float32),
        scratch_types=[
            pltpu.VMEM((NUM_BUCKETS, N_HEADS), jnp.float32),
            pltpu.VMEM((_NVALS + 1,), jnp.float32),
            pltpu.VMEM((8, _BW), jnp.float32),
            pltpu.VMEM((8, _BW), jnp.float32),
            pltpu.VMEM((8, _BW), jnp.float32),
            pltpu.SemaphoreType.DMA,
        ],
    )(table)


def kernel(seq_len, relative_attention_bias):
    del seq_len
    return _bias_sc(relative_attention_bias)


# speedup vs baseline: 142.4249x; 1.0050x over previous
import jax
import jax.numpy as jnp
from jax import lax
from jax.experimental import pallas as pl
from jax.experimental.pallas import tpu as pltpu
from jax.experimental.pallas import tpu_sc as plsc

N_HEADS = 16
NUM_BUCKETS = 32
SEQ_LEN = 2048
_D_MAX = SEQ_LEN - 1
_NVALS = 2 * SEQ_LEN - 1
_BW = 31 * 128
_THRESH = (12, 16, 23, 32, 46, 64, 91)
_N_CLASSES_PER_SUBCORE = 8


def _body(table_hbm, out_hbm, table_v, vals_v, b2a, b2b, b2c, dma_sem):
    core = lax.axis_index("c")
    sub = lax.axis_index("s")
    h = sub
    lane = lax.iota(jnp.int32, 16)

    pltpu.sync_copy(table_hbm, table_v)
    bufs = (b2a, b2b, b2c)

    def drain_class(buf):
        for _ in range(16):
            pltpu.make_async_copy(
                buf.at[:, pl.ds(0, SEQ_LEN)],
                out_hbm.at[0, pl.ds(0, 8), :],
                dma_sem).wait()

    def vals_chunk(cc, _):
        x = lane + cc * 16
        d = jnp.minimum(x, _NVALS - 1) - _D_MAX
        a = jnp.abs(d)
        big = jnp.full((16,), 8, jnp.int32)
        for t in _THRESH:
            big = big + (a >= t).astype(jnp.int32)
        bkt = jnp.where(a < 8, a, big) + jnp.where(d > 0, 16, 0)
        col = jnp.full((16,), h, jnp.int32)
        vals_v[pl.ds(cc * 16, 16)] = plsc.load_gather(table_v, [bkt, col])
        return 0

    lax.fori_loop(0, (_NVALS + 1) // 16, vals_chunk, 0, unroll=False)

    for k in range(_N_CLASSES_PER_SUBCORE):
        m = core * 8 + k
        qmin = 127 - 8 * m
        b2 = bufs[k % 3]

        if k >= 3:
            drain_class(b2)

        def b_chunk(cc, _, qmin=qmin, b2=b2):
            base = qmin + cc * 16
            for rr in range(8):
                b2[rr, pl.ds(cc * 16, 16)] = vals_v[pl.ds(base - rr, 16)]
            return 0

        lax.fori_loop(0, _BW // 16, b_chunk, 0, unroll=False)

        for u in range(16):
            co = 1920 - 128 * u
            pltpu.async_copy(
                b2.at[:, pl.ds(co, SEQ_LEN)],
                out_hbm.at[h, pl.ds(8 * m + 128 * u, 8), :],
                dma_sem)

    for buf in bufs:
        drain_class(buf)


@jax.jit
def _bias_sc(table):
    mesh = plsc.VectorSubcoreMesh(core_axis_name="c", subcore_axis_name="s")
    return pl.kernel(
        _body,
        mesh=mesh,
        compiler_params=pltpu.CompilerParams(needs_layout_passes=False),
        out_type=jax.ShapeDtypeStruct((N_HEADS, SEQ_LEN, SEQ_LEN),
                                      jnp.float32),
        scratch_types=[
            pltpu.VMEM((NUM_BUCKETS, N_HEADS), jnp.float32),
            pltpu.VMEM((_NVALS + 1,), jnp.float32),
            pltpu.VMEM((8, _BW), jnp.float32),
            pltpu.VMEM((8, _BW), jnp.float32),
            pltpu.VMEM((8, _BW), jnp.float32),
            pltpu.SemaphoreType.DMA,
        ],
    )(table)


def kernel(seq_len, relative_attention_bias):
    del seq_len
    return _bias_sc(relative_attention_bias)
